# baseline (speedup 1.0000x reference)
import functools

import jax
import jax.numpy as jnp
import numpy as np
from jax import lax
from jax.experimental import pallas as pl
from jax.experimental.pallas import tpu as pltpu
from jax.experimental.pallas import tpu_sc as plsc

M_MAX = 2
MAX_NO_ORBITALS_PER_M = 2
MAX_SPLIT_PER_M = 1
NCOEF = 4
NO_ORB = M_MAX * MAX_NO_ORBITALS_PER_M * MAX_SPLIT_PER_M
P = NO_ORB * (NO_ORB - 1) // 2
G = 16

NW = 32
CHUNK = 64
UNROLL = 8
PE = 8

_I_IDX, _J_IDX = np.triu_indices(NO_ORB, k=1)


def _emb_body(nblk_valid, x_ref, w_ref, b_ref, o_ref):
    i = pl.program_id(0)

    @pl.when(i < nblk_valid)
    def _():
        o_ref[...] = (
            jnp.dot(x_ref[...], w_ref[...], preferred_element_type=jnp.float32)
            + b_ref[...]
        )

    @pl.when(i >= nblk_valid)
    def _():
        o_ref[...] = jnp.zeros_like(o_ref)


def _make_emb(n_pairs, n_pad, emb_size):
    blk = 1280
    assert n_pairs % blk == 0 and n_pad % blk == 0
    nblk_valid = n_pairs // blk
    grid = n_pad // blk
    return pl.pallas_call(
        functools.partial(_emb_body, nblk_valid),
        grid=(grid,),
        in_specs=[
            pl.BlockSpec((blk, emb_size), lambda i: (jnp.minimum(i, nblk_valid - 1), 0)),
            pl.BlockSpec((emb_size, PE), lambda i: (0, 0)),
            pl.BlockSpec((1, PE), lambda i: (0, 0)),
        ],
        out_specs=pl.BlockSpec((blk, PE), lambda i: (i, 0)),
        out_shape=jax.ShapeDtypeStruct((n_pad, PE), jnp.float32),
    )


def _table_body(z_ref, r_ref, c_ref, o_ref):
    zf = z_ref[...]
    r = r_ref[...]
    coords = c_ref[...]
    d = r[:, None, :] - coords[None, :, :]
    dist = jnp.sqrt(jnp.sum(d * d, axis=-1))
    mult = 1.0 + 0.01 * jnp.sqrt(jnp.sum(r * r, axis=-1))
    o_i = lax.broadcasted_iota(jnp.int32, (NO_ORB, NCOEF), 0)
    c_i = lax.broadcasted_iota(jnp.int32, (NO_ORB, NCOEF), 1)
    arr = 0.5 + 0.1 * (NCOEF * o_i + c_i).astype(jnp.float32)
    base = 0.1 * (zf[:, 0] + 1.0) * mult
    coeff = base[:, None, None] * arr[None, :, :]
    earg = coeff[:, :, None, :] * dist[:, None, :, None]
    orb = jnp.sum(coeff[:, :, None, :] * jnp.exp(-earg), axis=-1)
    chans = [orb[:, int(i), :] * orb[:, int(j), :] for i, j in zip(_I_IDX, _J_IDX)]
    o_ref[...] = jnp.concatenate(chans, axis=-1)


def _make_table(n_atoms):
    blk = 400
    assert n_atoms % blk == 0
    grid = n_atoms // blk
    return pl.pallas_call(
        _table_body,
        grid=(grid,),
        in_specs=[
            pl.BlockSpec((blk, 1), lambda i: (i, 0)),
            pl.BlockSpec((blk, 3), lambda i: (i, 0)),
            pl.BlockSpec((G, 3), lambda i: (0, 0)),
        ],
        out_specs=pl.BlockSpec((blk, P * G), lambda i: (i, 0)),
        out_shape=jax.ShapeDtypeStruct((n_atoms, P * G), jnp.float32),
    )


DEPTH = 4
SPLIT = 4
RPS = 2 * CHUNK // SPLIT


def _sc_body(n_chunks, n_mol, table_hbm, idx_hbm, mol_hbm, emb_hbm, out_hbm,
             idx_v, mol_v,
             emb_0, emb_1, emb_2, emb_3,
             rows_0, rows_1, rows_2, rows_3,
             acc_v, sem_0, sem_1, sem_2, sem_3):
    wid = lax.axis_index("s") * 2 + lax.axis_index("c")
    emb_l = [emb_0, emb_1, emb_2, emb_3]
    rows_l = [rows_0, rows_1, rows_2, rows_3]
    sem_l = [sem_0, sem_1, sem_2, sem_3]

    tp = n_chunks * CHUNK
    pltpu.sync_copy(idx_hbm.at[wid], idx_v)
    pltpu.sync_copy(mol_hbm.at[wid], mol_v.at[pl.ds(0, tp)])

    zero = jnp.zeros((16,), jnp.float32)

    def zinit(m, carry):
        for p in range(P):
            acc_v[m, pl.ds(16 * p, 16)] = zero
        return carry

    lax.fori_loop(0, n_mol, zinit, 0)

    def start(ci, b):
        for s in range(SPLIT):
            pltpu.async_copy(
                table_hbm.at[idx_v.at[ci, pl.ds(s * RPS, RPS)]],
                rows_l[b].at[pl.ds(s * RPS, RPS)],
                sem_l[b],
            )
        pltpu.async_copy(
            emb_hbm.at[wid].at[ci], emb_l[b].at[pl.ds(0, CHUNK * PE)], sem_l[b]
        )

    def wait(b):
        for s in range(SPLIT):
            pltpu.make_async_copy(
                table_hbm.at[idx_v.at[0, pl.ds(0, RPS)]],
                rows_l[b].at[pl.ds(s * RPS, RPS)],
                sem_l[b],
            ).wait()
        pltpu.make_async_copy(
            emb_hbm.at[wid].at[0], emb_l[b].at[pl.ds(0, CHUNK * PE)], sem_l[b]
        ).wait()

    def compute(ci, rows_v, emb_v):
        g0 = ci * CHUNK
        mol_lo = mol_v[pl.ds(g0, 16)][0]
        mol_hi = mol_v[pl.ds(g0 + CHUNK - 1, 16)][0]

        def one_pair(k, accs):
            e_vec = emb_v[pl.ds(k * PE, 16)]
            o = []
            for p in range(P):
                vi = rows_v[2 * k, pl.ds(16 * p, 16)]
                vj = rows_v[2 * k + 1, pl.ds(16 * p, 16)]
                o.append(accs[p] + vi * vj * e_vec[p])
            return tuple(o)

        def fast(_):
            def grp(t, accs):
                for u in range(UNROLL):
                    accs = one_pair(t * UNROLL + u, accs)
                return accs

            accs = lax.fori_loop(0, CHUNK // UNROLL, grp, (zero,) * P)
            for p in range(P):
                acc_v[mol_lo, pl.ds(16 * p, 16)] += accs[p]
            return 0

        def slow(_):
            def pair_body(k, c2):
                g = g0 + k
                mol = mol_v[pl.ds(g, 16)][0]
                e_vec = emb_v[pl.ds(k * PE, 16)]
                for p in range(P):
                    vi = rows_v[2 * k, pl.ds(16 * p, 16)]
                    vj = rows_v[2 * k + 1, pl.ds(16 * p, 16)]
                    acc_v[mol, pl.ds(16 * p, 16)] += vi * vj * e_vec[p]
                return c2

            lax.fori_loop(0, CHUNK, pair_body, 0)
            return 0

        lax.cond(mol_lo == mol_hi, fast, slow, 0)

    for d in range(DEPTH - 1):
        start(d, d)

    def ring(t, carry):
        ci0 = DEPTH * t
        for u in range(DEPTH):
            cj = ci0 + u

            @pl.when(cj + DEPTH - 1 < n_chunks)
            def _(cj=cj, b=(u + DEPTH - 1) % DEPTH):
                start(cj + DEPTH - 1, b)

            wait(u)
            compute(cj, rows_l[u], emb_l[u])
        return carry

    lax.fori_loop(0, n_chunks // DEPTH, ring, 0)
    pltpu.sync_copy(acc_v, out_hbm.at[wid])


def _make_sc(n_atoms, n_chunks, n_mol):
    tp = n_chunks * CHUNK
    mesh = plsc.VectorSubcoreMesh(
        core_axis_name="c", subcore_axis_name="s", num_cores=2, num_subcores=16
    )
    assert n_chunks % DEPTH == 0
    return pl.kernel(
        functools.partial(_sc_body, n_chunks, n_mol),
        out_type=jax.ShapeDtypeStruct((NW, n_mol, P * G), jnp.float32),
        mesh=mesh,
        scratch_types=[
            pltpu.VMEM((n_chunks, 2 * CHUNK), jnp.int32),
            pltpu.VMEM((tp + 16,), jnp.int32),
        ]
        + [pltpu.VMEM((CHUNK * PE + 16,), jnp.float32)] * DEPTH
        + [pltpu.VMEM((2 * CHUNK, P * G), jnp.float32)] * DEPTH
        + [pltpu.VMEM((n_mol, P * G), jnp.float32)]
        + [pltpu.SemaphoreType.DMA] * DEPTH,
        compiler_params=pltpu.CompilerParams(use_tc_tiling_on_sc=False),
    )


def _reduce_body(p_ref, o_ref):
    o_ref[...] = jnp.sum(p_ref[...], axis=0)


def _make_reduce(n_mol):
    return pl.pallas_call(
        _reduce_body,
        out_shape=jax.ShapeDtypeStruct((n_mol, P * G), jnp.float32),
    )


def kernel(out, Z, R, coords, N, atom_pair_indices, atom_pair_mol_id, W, b):
    n_pairs, emb_size = out.shape
    n_atoms = Z.shape[0]
    n_mol = N.shape[0]

    round_to = 40960
    n_pad = ((n_pairs + round_to - 1) // round_to) * round_to
    tp = n_pad // NW
    n_chunks = tp // CHUNK

    w_pad = jnp.zeros((emb_size, PE), jnp.float32).at[:, :P].set(W)
    b_pad = jnp.zeros((1, PE), jnp.float32).at[0, :P].set(b)

    emb = _make_emb(n_pairs, n_pad, emb_size)(out, w_pad, b_pad)
    table = _make_table(n_atoms)(
        Z.astype(jnp.float32).reshape(n_atoms, 1), R, coords
    )

    pad_n = n_pad - n_pairs
    idx = jnp.pad(atom_pair_indices.reshape(-1), (0, 2 * pad_n)).reshape(
        NW, n_chunks, 2 * CHUNK
    )
    mol = jnp.pad(atom_pair_mol_id, (0, pad_n)).reshape(NW, tp)
    embr = emb.reshape(NW, tp // CHUNK, CHUNK * PE)

    partials = _make_sc(n_atoms, n_chunks, n_mol)(table, idx, mol, embr)
    dens = _make_reduce(n_mol)(partials)
    return jnp.transpose(dens.reshape(n_mol, P, G), (0, 2, 1))


# speedup vs baseline: 1.2075x; 1.2075x over previous
import functools

import jax
import jax.numpy as jnp
import numpy as np
from jax import lax
from jax.experimental import pallas as pl
from jax.experimental.pallas import tpu as pltpu
from jax.experimental.pallas import tpu_sc as plsc

M_MAX = 2
MAX_NO_ORBITALS_PER_M = 2
MAX_SPLIT_PER_M = 1
NCOEF = 4
NO_ORB = M_MAX * MAX_NO_ORBITALS_PER_M * MAX_SPLIT_PER_M
P = NO_ORB * (NO_ORB - 1) // 2
G = 16

NW = 32
CHUNK = 64
UNROLL = 8
PE = 8

_I_IDX, _J_IDX = np.triu_indices(NO_ORB, k=1)


def _emb_body(nblk_valid, x_ref, w_ref, b_ref, o_ref):
    i = pl.program_id(0)

    @pl.when(i < nblk_valid)
    def _():
        o_ref[...] = (
            jnp.dot(x_ref[...], w_ref[...], preferred_element_type=jnp.float32)
            + b_ref[...]
        )

    @pl.when(i >= nblk_valid)
    def _():
        o_ref[...] = jnp.zeros_like(o_ref)


def _make_emb(n_pairs, n_pad, emb_size):
    blk = 1280
    assert n_pairs % blk == 0 and n_pad % blk == 0
    nblk_valid = n_pairs // blk
    grid = n_pad // blk
    return pl.pallas_call(
        functools.partial(_emb_body, nblk_valid),
        grid=(grid,),
        in_specs=[
            pl.BlockSpec((blk, emb_size), lambda i: (jnp.minimum(i, nblk_valid - 1), 0)),
            pl.BlockSpec((emb_size, PE), lambda i: (0, 0)),
            pl.BlockSpec((1, PE), lambda i: (0, 0)),
        ],
        out_specs=pl.BlockSpec((blk, PE), lambda i: (i, 0)),
        out_shape=jax.ShapeDtypeStruct((n_pad, PE), jnp.float32),
    )


def _table_body(z_ref, r_ref, c_ref, o_ref):
    zf = z_ref[...]
    r = r_ref[...]
    coords = c_ref[...]
    d = r[:, None, :] - coords[None, :, :]
    dist = jnp.sqrt(jnp.sum(d * d, axis=-1))
    mult = 1.0 + 0.01 * jnp.sqrt(jnp.sum(r * r, axis=-1))
    o_i = lax.broadcasted_iota(jnp.int32, (NO_ORB, NCOEF), 0)
    c_i = lax.broadcasted_iota(jnp.int32, (NO_ORB, NCOEF), 1)
    arr = 0.5 + 0.1 * (NCOEF * o_i + c_i).astype(jnp.float32)
    base = 0.1 * (zf[:, 0] + 1.0) * mult
    coeff = base[:, None, None] * arr[None, :, :]
    earg = coeff[:, :, None, :] * dist[:, None, :, None]
    orb = jnp.sum(coeff[:, :, None, :] * jnp.exp(-earg), axis=-1)
    chans = [orb[:, int(i), :] * orb[:, int(j), :] for i, j in zip(_I_IDX, _J_IDX)]
    o_ref[...] = jnp.concatenate(chans, axis=-1)


def _make_table(n_atoms):
    blk = 400
    assert n_atoms % blk == 0
    grid = n_atoms // blk
    return pl.pallas_call(
        _table_body,
        grid=(grid,),
        in_specs=[
            pl.BlockSpec((blk, 1), lambda i: (i, 0)),
            pl.BlockSpec((blk, 3), lambda i: (i, 0)),
            pl.BlockSpec((G, 3), lambda i: (0, 0)),
        ],
        out_specs=pl.BlockSpec((blk, P * G), lambda i: (i, 0)),
        out_shape=jax.ShapeDtypeStruct((n_atoms, P * G), jnp.float32),
    )


def _sc_body(n_chunks, n_mol, n_atoms, table_hbm, idx_hbm, mol_hbm, emb_hbm,
             out_hbm, table_sh, idx_v, mol_v, emb_v, rows_v, acc_v, sem):
    sid = lax.axis_index("s")
    wid = sid * 2 + lax.axis_index("c")

    rows_per = n_atoms // 16
    pltpu.sync_copy(
        table_hbm.at[pl.ds(sid * rows_per, rows_per)],
        table_sh.at[pl.ds(sid * rows_per, rows_per)],
    )

    tp = n_chunks * CHUNK
    pltpu.sync_copy(idx_hbm.at[wid], idx_v)
    pltpu.sync_copy(mol_hbm.at[wid], mol_v.at[pl.ds(0, tp)])
    plsc.subcore_barrier()

    zero = jnp.zeros((16,), jnp.float32)

    def zinit(m, carry):
        for p in range(P):
            acc_v[m, pl.ds(16 * p, 16)] = zero
        return carry

    lax.fori_loop(0, n_mol, zinit, 0)

    def compute(ci, rows_v, emb_v):
        g0 = ci * CHUNK
        mol_lo = mol_v[pl.ds(g0, 16)][0]
        mol_hi = mol_v[pl.ds(g0 + CHUNK - 1, 16)][0]

        def one_pair(k, accs):
            e_vec = emb_v[pl.ds(k * PE, 16)]
            o = []
            for p in range(P):
                vi = rows_v[2 * k, pl.ds(16 * p, 16)]
                vj = rows_v[2 * k + 1, pl.ds(16 * p, 16)]
                o.append(accs[p] + vi * vj * e_vec[p])
            return tuple(o)

        def fast(_):
            def grp(t, accs):
                for u in range(UNROLL):
                    accs = one_pair(t * UNROLL + u, accs)
                return accs

            accs = lax.fori_loop(0, CHUNK // UNROLL, grp, (zero,) * P)
            for p in range(P):
                acc_v[mol_lo, pl.ds(16 * p, 16)] += accs[p]
            return 0

        def slow(_):
            def pair_body(k, c2):
                g = g0 + k
                mol = mol_v[pl.ds(g, 16)][0]
                e_vec = emb_v[pl.ds(k * PE, 16)]
                for p in range(P):
                    vi = rows_v[2 * k, pl.ds(16 * p, 16)]
                    vj = rows_v[2 * k + 1, pl.ds(16 * p, 16)]
                    acc_v[mol, pl.ds(16 * p, 16)] += vi * vj * e_vec[p]
                return c2

            lax.fori_loop(0, CHUNK, pair_body, 0)
            return 0

        lax.cond(mol_lo == mol_hi, fast, slow, 0)

    def chunk_body(ci, carry):
        pltpu.async_copy(table_sh.at[idx_v.at[ci]], rows_v, sem).wait()
        pltpu.async_copy(
            emb_hbm.at[wid].at[ci], emb_v.at[pl.ds(0, CHUNK * PE)], sem
        ).wait()
        compute(ci, rows_v, emb_v)
        return carry

    lax.fori_loop(0, n_chunks, chunk_body, 0)
    pltpu.sync_copy(acc_v, out_hbm.at[wid])


def _make_sc(n_atoms, n_chunks, n_mol):
    tp = n_chunks * CHUNK
    mesh = plsc.VectorSubcoreMesh(
        core_axis_name="c", subcore_axis_name="s", num_cores=2, num_subcores=16
    )
    assert n_atoms % 16 == 0
    return pl.kernel(
        functools.partial(_sc_body, n_chunks, n_mol, n_atoms),
        out_type=jax.ShapeDtypeStruct((NW, n_mol, P * G), jnp.float32),
        mesh=mesh,
        scratch_types=[
            pltpu.VMEM_SHARED((n_atoms, P * G), jnp.float32),
            pltpu.VMEM((n_chunks, 2 * CHUNK), jnp.int32),
            pltpu.VMEM((tp + 16,), jnp.int32),
            pltpu.VMEM((CHUNK * PE + 16,), jnp.float32),
            pltpu.VMEM((2 * CHUNK, P * G), jnp.float32),
            pltpu.VMEM((n_mol, P * G), jnp.float32),
            pltpu.SemaphoreType.DMA,
        ],
        compiler_params=pltpu.CompilerParams(use_tc_tiling_on_sc=False),
    )


def _reduce_body(p_ref, o_ref):
    o_ref[...] = jnp.sum(p_ref[...], axis=0)


def _make_reduce(n_mol):
    return pl.pallas_call(
        _reduce_body,
        out_shape=jax.ShapeDtypeStruct((n_mol, P * G), jnp.float32),
    )


def kernel(out, Z, R, coords, N, atom_pair_indices, atom_pair_mol_id, W, b):
    n_pairs, emb_size = out.shape
    n_atoms = Z.shape[0]
    n_mol = N.shape[0]

    round_to = 40960
    n_pad = ((n_pairs + round_to - 1) // round_to) * round_to
    tp = n_pad // NW
    n_chunks = tp // CHUNK

    w_pad = jnp.zeros((emb_size, PE), jnp.float32).at[:, :P].set(W)
    b_pad = jnp.zeros((1, PE), jnp.float32).at[0, :P].set(b)

    emb = _make_emb(n_pairs, n_pad, emb_size)(out, w_pad, b_pad)
    table = _make_table(n_atoms)(
        Z.astype(jnp.float32).reshape(n_atoms, 1), R, coords
    )

    pad_n = n_pad - n_pairs
    idx = jnp.pad(atom_pair_indices.reshape(-1), (0, 2 * pad_n)).reshape(
        NW, n_chunks, 2 * CHUNK
    )
    mol = jnp.pad(atom_pair_mol_id, (0, pad_n)).reshape(NW, tp)
    embr = emb.reshape(NW, tp // CHUNK, CHUNK * PE)

    partials = _make_sc(n_atoms, n_chunks, n_mol)(table, idx, mol, embr)
    dens = _make_reduce(n_mol)(partials)
    return jnp.transpose(dens.reshape(n_mol, P, G), (0, 2, 1))
